# Initial kernel scaffold; baseline (speedup 1.0000x reference)
import jax
import jax.numpy as jnp
from jax.experimental import pallas as pl


def kernel(x, edge_index, edge_attr, edge_weight, batch, params):
    raise NotImplementedError("write your pallas kernel here")


# speedup vs baseline: 2.4395x; 2.4395x over previous
import functools

import jax
import jax.numpy as jnp
from jax import lax
from jax.experimental import pallas as pl
from jax.experimental.pallas import tpu as pltpu
from jax.experimental.pallas import tpu_sc as plsc

N_NODES = 10000
N_EDGES = 320000
D_FEAT = 128
D_EDGE = 16
DIM = 128
NUM_CLASS = 12
NUM_GRAPHS = 64
STEPS = 6

NC = 2
NS = 16
NW = NC * NS
CH = 128
N_CHUNKS = N_EDGES // CH
CHUNKS_PER_W = -(-N_CHUNKS // NW)
ZROWS = 80
N_ZCH = N_NODES // ZROWS
ZITER = -(-N_ZCH // NS)
LG = DIM // 16


def _edge_body(h_hbm, e_hbm, src_hbm, dst_hbm, w_hbm, out_hbm,
               idx_v, dst_v, w_v, rows_v, e_v, zero_v, agg_sh, sem):
  cid = lax.axis_index("c")
  sid = lax.axis_index("s")
  wid = sid * NC + cid

  def zfill(i, _):
    zero_v[i // LG, pl.ds((i % LG) * 16, 16)] = jnp.zeros((16,), jnp.float32)
    return 0
  lax.fori_loop(0, ZROWS * LG, zfill, 0)
  for z in range(ZITER):
    idx = sid + z * NS
    @pl.when(idx < N_ZCH)
    def _():
      pltpu.sync_copy(zero_v, agg_sh.at[pl.ds(idx * ZROWS, ZROWS)])
  plsc.subcore_barrier()

  def chunk_body(k, _):
    c = wid + k * NW

    @pl.when(c < N_CHUNKS)
    def _():
      base = c * CH
      pltpu.sync_copy(src_hbm.at[pl.ds(base, CH)], idx_v)
      pltpu.sync_copy(dst_hbm.at[pl.ds(base, CH)], dst_v)
      pltpu.sync_copy(w_hbm.at[pl.ds(base, CH)], w_v)
      pltpu.sync_copy(e_hbm.at[pl.ds(base, CH)], e_v)
      pltpu.async_copy(h_hbm.at[idx_v], rows_v, sem).wait()

      def edge_group(q, _):
        wvec = w_v[pl.ds(q * 16, 16)]
        for t in range(16):
          wj = wvec[t]
          j = q * 16 + t
          for g in range(LG):
            sl = pl.ds(g * 16, 16)
            rows_v[j, sl] = (
                jnp.maximum(rows_v[j, sl] + e_v[j, sl], 0.0) * wj)
        return 0
      lax.fori_loop(0, CH // 16, edge_group, 0)
      pltpu.sync_copy(rows_v, agg_sh.at[dst_v], add=True)
    return 0

  lax.fori_loop(0, CHUNKS_PER_W, chunk_body, 0)
  plsc.subcore_barrier()
  for z in range(ZITER):
    idx = sid + z * NS
    @pl.when(idx < N_ZCH)
    def _():
      pltpu.sync_copy(agg_sh.at[pl.ds(idx * ZROWS, ZROWS)],
                      out_hbm.at[cid, pl.ds(idx * ZROWS, ZROWS)])


_edge_call_cached = None


def _edge_call(h, e, src, dst, w):
  global _edge_call_cached
  if _edge_call_cached is None:
    _edge_call_cached = pl.kernel(
        _edge_body,
        out_type=jax.ShapeDtypeStruct((NC, N_NODES, DIM), jnp.float32),
        mesh=plsc.VectorSubcoreMesh(core_axis_name="c", subcore_axis_name="s",
                                    num_cores=NC, num_subcores=NS),
        scratch_types=[
            pltpu.VMEM((CH,), jnp.int32),
            pltpu.VMEM((CH,), jnp.int32),
            pltpu.VMEM((CH,), jnp.float32),
            pltpu.VMEM((CH, DIM), jnp.float32),
            pltpu.VMEM((CH, DIM), jnp.float32),
            pltpu.VMEM((ZROWS, DIM), jnp.float32),
            pltpu.VMEM_SHARED((N_NODES, DIM), jnp.float32),
            pltpu.SemaphoreType.DMA,
        ],
    )
  return _edge_call_cached(h, e, src, dst, w)


_BOND_BLK = 4000


def _bond_body(ea_ref, w_ref, b_ref, out_ref):
  out_ref[...] = jnp.dot(ea_ref[...], w_ref[...],
                         preferred_element_type=jnp.float32) + b_ref[...]


def _bond_call(edge_attr, w, b):
  grid = (N_EDGES // _BOND_BLK,)
  return pl.pallas_call(
      _bond_body,
      grid=grid,
      in_specs=[
          pl.BlockSpec((_BOND_BLK, D_EDGE), lambda i: (i, 0)),
          pl.BlockSpec((D_EDGE, DIM), lambda i: (0, 0)),
          pl.BlockSpec((1, DIM), lambda i: (0, 0)),
      ],
      out_specs=pl.BlockSpec((_BOND_BLK, DIM), lambda i: (i, 0)),
      out_shape=jax.ShapeDtypeStruct((N_EDGES, DIM), jnp.float32),
  )(edge_attr, w, b[None, :])


_MLP_BLK = 1000


def _mlp_body(h_ref, a0_ref, a1_ref, w1_ref, b1_ref, w2_ref, b2_ref, o_ref):
  z = h_ref[...] + a0_ref[...] + a1_ref[...]
  t = jnp.maximum(
      jnp.dot(z, w1_ref[...], preferred_element_type=jnp.float32)
      + b1_ref[...], 0.0)
  y = (jnp.dot(t, w2_ref[...], preferred_element_type=jnp.float32)
       + b2_ref[...])
  o_ref[...] = jnp.maximum(y, 0.0)


def _mlp_call(h, a0, a1, w1, b1, w2, b2):
  grid = (N_NODES // _MLP_BLK,)
  blk = lambda: pl.BlockSpec((_MLP_BLK, DIM), lambda i: (i, 0))
  wspec = lambda: pl.BlockSpec((DIM, DIM), lambda i: (0, 0))
  bspec = lambda: pl.BlockSpec((1, DIM), lambda i: (0, 0))
  return pl.pallas_call(
      _mlp_body,
      grid=grid,
      in_specs=[blk(), blk(), blk(), wspec(), bspec(), wspec(), bspec()],
      out_specs=blk(),
      out_shape=jax.ShapeDtypeStruct((N_NODES, DIM), jnp.float32),
  )(h, a0, a1, w1, b1[None, :], w2, b2[None, :])


NP = 10240


def _s2s_body(x_ref, b_ref, wih_ref, whh_ref, bg_ref, fc1w_ref, fc1b_ref,
              fc4w_ref, fc4b_ref, out_ref):
  x = x_ref[...]
  bat = b_ref[...][0:1, :]
  gids = lax.broadcasted_iota(jnp.int32, (NUM_GRAPHS, NP), 0)
  onehot_b = jnp.broadcast_to(bat, (NUM_GRAPHS, NP)) == gids
  wih = wih_ref[...]
  whh = whh_ref[...]

  hh = jnp.zeros((NUM_GRAPHS, DIM), jnp.float32)
  cc = jnp.zeros((NUM_GRAPHS, DIM), jnp.float32)
  q_star = jnp.zeros((NUM_GRAPHS, 2 * DIM), jnp.float32)
  nt = (((1,), (1,)), ((), ()))
  for _ in range(STEPS):
    gates = (lax.dot_general(q_star, wih, nt,
                             preferred_element_type=jnp.float32)
             + lax.dot_general(hh, whh, nt,
                               preferred_element_type=jnp.float32)
             + bg_ref[...])
    ig = jax.nn.sigmoid(gates[:, 0:DIM])
    fg = jax.nn.sigmoid(gates[:, DIM:2 * DIM])
    gg = jnp.tanh(gates[:, 2 * DIM:3 * DIM])
    og = jax.nn.sigmoid(gates[:, 3 * DIM:4 * DIM])
    cc = fg * cc + ig * gg
    hh = og * jnp.tanh(cc)
    xq = lax.dot_general(hh, x, nt, preferred_element_type=jnp.float32)
    e_row = jnp.sum(jnp.where(onehot_b, xq, 0.0), axis=0, keepdims=True)
    e_b = jnp.broadcast_to(e_row, (NUM_GRAPHS, NP))
    e_max = jnp.max(jnp.where(onehot_b, e_b, -jnp.inf), axis=1,
                    keepdims=True)
    e_max = jnp.where(e_max > -1e30, e_max, 0.0)
    ee = jnp.where(onehot_b,
                   jnp.exp(e_b - jnp.broadcast_to(e_max, (NUM_GRAPHS, NP))),
                   0.0)
    denom = jnp.sum(ee, axis=1, keepdims=True)
    a = ee / (jnp.broadcast_to(denom, (NUM_GRAPHS, NP)) + 1e-16)
    r = jnp.dot(a, x, preferred_element_type=jnp.float32)
    q_star = jnp.concatenate([hh, r], axis=1)

  o1 = jnp.maximum(
      jnp.dot(q_star, fc1w_ref[...], preferred_element_type=jnp.float32)
      + fc1b_ref[...], 0.0)
  out_ref[...] = (jnp.dot(o1, fc4w_ref[...],
                          preferred_element_type=jnp.float32)
                  + fc4b_ref[...])


def _s2s_call(xp, b8, wih, whh, bg, fc1w, fc1b, fc4wp, fc4bp):
  return pl.pallas_call(
      _s2s_body,
      out_shape=jax.ShapeDtypeStruct((NUM_GRAPHS, DIM), jnp.float32),
  )(xp, b8, wih, whh, bg[None, :], fc1w, fc1b[None, :], fc4wp, fc4bp[None, :])


@jax.jit
def _run(x, edge_index, edge_attr, edge_weight, batch, params):
  src = edge_index[0]
  dst = edge_index[1]
  h = x
  for l in range(6):
    p = params['l%d' % l]
    e = _bond_call(edge_attr, p['bond_W'], p['bond_b'])
    agg = _edge_call(h, e, src, dst, edge_weight)
    h = _mlp_call(h, agg[0], agg[1], p['W1'], p['b1'], p['W2'], p['b2'])

  xp = jnp.pad(h, ((0, NP - N_NODES), (0, 0)))
  batch_pad = jnp.concatenate(
      [batch, jnp.full((NP - N_NODES,), NUM_GRAPHS, jnp.int32)])
  b8 = jnp.broadcast_to(batch_pad[None, :], (8, NP))
  s2s = params['s2s']
  fc4wp = jnp.pad(params['fc4_W'], ((0, 0), (0, DIM - NUM_CLASS)))
  fc4bp = jnp.pad(params['fc4_b'], (0, DIM - NUM_CLASS))
  bg = s2s['b_ih'] + s2s['b_hh']
  out = _s2s_call(xp, b8, s2s['W_ih'], s2s['W_hh'], bg, params['fc1_W'],
                  params['fc1_b'], fc4wp, fc4bp)
  return out[:, :NUM_CLASS]


def kernel(x, edge_index, edge_attr, edge_weight, batch, params):
  return _run(x, edge_index, edge_attr, edge_weight, batch, params)
